# baseline (speedup 1.0000x reference)
import functools
import math

import jax
import jax.numpy as jnp
from jax.experimental import pallas as pl
from jax.experimental.pallas import tpu as pltpu

_SH = 7


def _sel(table, idx):
    v = jnp.int32(table[0])
    for k in range(1, len(table)):
        v = jnp.where(idx >= k, jnp.int32(table[k]), v)
    return v


def _big_body(H, in_dim, Bp, tbl, tail_step, n_steps,
              embed, ss_h, sc_h, xt, sst, sct, xpt,
              R_pair, W_iou, U_iou, b_iou, W_f, U_f, b_f,
              dh_fin, dc_fin, hacc_out,
              x_buf, ss_buf, sc_buf, xp_buf, dh_buf, dc_buf, sems):
    B2 = 2 * Bp

    def params(step):
        lv = jnp.int32(0)
        for st in tbl["start"][1:]:
            lv = lv + jnp.where(step >= st, 1, 0).astype(jnp.int32)
        j = step - _sel(tbl["start"], lv)
        return lv, j

    def copy_in(step, slot):
        lv, j = params(step)
        cb8 = _sel(tbl["s8"], lv) + j * (B2 // 8)
        pb8 = _sel(tbl["sp8"], lv) + j * (Bp // 8)
        cb = pl.multiple_of(cb8 * 8, 8)
        pb = pl.multiple_of(pb8 * 8, 8)

        @pl.when(step != tail_step)
        def _():
            pltpu.make_async_copy(embed.at[pl.ds(cb, B2 + 8), :],
                                  x_buf.at[slot], sems.at[slot, 0]).start()
            pltpu.make_async_copy(ss_h.at[pl.ds(cb, B2 + 8), :],
                                  ss_buf.at[slot], sems.at[slot, 1]).start()
            pltpu.make_async_copy(sc_h.at[pl.ds(cb, B2 + 8), :],
                                  sc_buf.at[slot], sems.at[slot, 2]).start()
            pltpu.make_async_copy(embed.at[pl.ds(pb, Bp + 8), :],
                                  xp_buf.at[slot], sems.at[slot, 3]).start()

        @pl.when(step == tail_step)
        def _():
            pltpu.make_async_copy(xt.at[:, :], x_buf.at[slot],
                                  sems.at[slot, 0]).start()
            pltpu.make_async_copy(sst.at[:, :], ss_buf.at[slot],
                                  sems.at[slot, 1]).start()
            pltpu.make_async_copy(sct.at[:, :], sc_buf.at[slot],
                                  sems.at[slot, 2]).start()
            pltpu.make_async_copy(xpt.at[:, :], xp_buf.at[slot],
                                  sems.at[slot, 3]).start()

    def wait_one(slot, k, buf, ref, rows_):
        pltpu.make_async_copy(ref.at[pl.ds(0, rows_), :],
                              buf.at[slot], sems.at[slot, k]).wait()

    i = pl.program_id(0)
    slot = jax.lax.rem(i, 3)

    @pl.when(i == 0)
    def _():
        copy_in(0, 0)
        copy_in(1, 1)

    @pl.when(i + 2 < n_steps)
    def _():
        copy_in(i + 2, jax.lax.rem(i + 2, 3))

    lv, j = params(i)
    p = jax.lax.rem(lv, 2)
    L_rem = _sel(tbl["L"], lv) - j * B2
    C_rem = _sel(tbl["C"], lv) - j * B2
    rd8 = jnp.minimum(j * (B2 // 8), _sel(tbl["cap8"], lv))
    rd = pl.multiple_of(rd8 * 8, 8)
    wr = pl.multiple_of(j * Bp, Bp)

    rows = jax.lax.broadcasted_iota(jnp.int32, (B2, 1), 0)
    m = rows < L_rem
    m_c = rows < C_rem

    wait_one(slot, 0, x_buf, embed, B2 + 8)
    x = x_buf[slot][_SH:_SH + B2]
    xW = jnp.dot(x, W_iou[:, :], preferred_element_type=jnp.float32)

    wait_one(slot, 1, ss_buf, ss_h, B2 + 8)
    wait_one(slot, 2, sc_buf, sc_h, B2 + 8)
    hsum = ss_buf[slot][_SH:_SH + B2]
    cin = sc_buf[slot][_SH:_SH + B2]
    hsum = hsum + jnp.where(m_c, dh_buf[1 - p, pl.ds(rd, B2), :], 0.0)
    cin = cin + jnp.where(m_c, dc_buf[1 - p, pl.ds(rd, B2), :], 0.0)
    iou = (xW + b_iou[:, :]
           + jnp.dot(hsum, U_iou[:, :], preferred_element_type=jnp.float32))
    c = jax.nn.sigmoid(iou[:, :H]) * jnp.tanh(iou[:, 2 * H:]) + cin
    h = jax.nn.sigmoid(iou[:, H:2 * H]) * jnp.tanh(c)
    h = jnp.where(m, h, 0.0)

    wait_one(slot, 3, xp_buf, embed, Bp + 8)
    xpW = jnp.dot(xp_buf[slot][_SH:_SH + Bp], W_f[:, :],
                  preferred_element_type=jnp.float32) + b_f[:, :]
    xpW2 = jnp.broadcast_to(xpW[:, None, :], (Bp, 2, H)).reshape(B2, H)
    f = jax.nn.sigmoid(xpW2 + jnp.dot(h, U_f[:, :],
                                      preferred_element_type=jnp.float32))
    fc = jnp.where(m, f * c, 0.0)
    dh_t = jnp.dot(R_pair[:, :], h, preferred_element_type=jnp.float32)
    dc_t = jnp.dot(R_pair[:, :], fc, preferred_element_type=jnp.float32)
    dh_buf[p, pl.ds(wr, Bp), :] = dh_t
    dc_buf[p, pl.ds(wr, Bp), :] = dc_t
    part = jnp.sum(h, axis=0, keepdims=True)

    @pl.when(i == 0)
    def _():
        hacc_out[:, :] = part

    @pl.when(i > 0)
    def _():
        hacc_out[:, :] = hacc_out[:, :] + part

    @pl.when(i == n_steps - 1)
    def _():
        dh_fin[:, :] = dh_t
        dc_fin[:, :] = dc_t


def _top_body(H, top_d, x_all, ss_all, sc_all, dh_in, dc_in,
              W_iou, U_iou, b_iou, W_f, U_f, b_f, hacc_in, mu_out, lv_out):
    Wi = W_iou[:, :]
    Ui = U_iou[:, :]
    bi = b_iou[:, :]
    Wf = W_f[:, :]
    Uf = U_f[:, :]
    bf = b_f[:, :]
    xv = x_all[:, :]
    ssv = ss_all[:, :]
    scv = sc_all[:, :]
    hacc = hacc_in[:, :]
    dh = dh_in[:, :]
    dc = dc_in[:, :]

    for d in range(top_d, -1, -1):
        s = 2 ** d - 1
        L = 2 ** d
        x_l = xv[s:s + L]
        hs = ssv[s:s + L] + dh[:L]
        cn = scv[s:s + L] + dc[:L]
        iou = (jnp.dot(x_l, Wi, preferred_element_type=jnp.float32) + bi
               + jnp.dot(hs, Ui, preferred_element_type=jnp.float32))
        c = jax.nn.sigmoid(iou[:, :H]) * jnp.tanh(iou[:, 2 * H:]) + cn
        h = jax.nn.sigmoid(iou[:, H:2 * H]) * jnp.tanh(c)
        hacc = hacc + jnp.sum(h, axis=0, keepdims=True)
        if d == 0:
            break
        Lp = L // 2
        sp = 2 ** (d - 1) - 1
        xpW = jnp.dot(xv[sp:sp + Lp], Wf,
                      preferred_element_type=jnp.float32) + bf
        xpW2 = jnp.broadcast_to(xpW[:, None, :], (Lp, 2, H)).reshape(L, H)
        f = jax.nn.sigmoid(xpW2 + jnp.dot(h, Uf,
                                          preferred_element_type=jnp.float32))
        dh = h.reshape(Lp, 2, H).sum(axis=1)
        dc = (f * c).reshape(Lp, 2, H).sum(axis=1)

    mu_out[:, :] = hacc[:, :H // 2]
    lv_out[:, :] = jnp.tanh(hacc[:, H // 2:])


def _pad_rows(a, rows):
    if a.shape[0] == rows:
        return a
    return jnp.pad(a, ((0, rows - a.shape[0]), (0, 0)))


def kernel(embed, edge_index, structure_sum, structure_c,
           W_iou, U_iou, b_iou, W_f, U_f, b_f):
    del edge_index
    n = embed.shape[0]
    in_dim = embed.shape[1]
    H = U_f.shape[0]
    f32 = jnp.float32

    b_iou2 = b_iou.reshape(1, 3 * H)
    b_f2 = b_f.reshape(1, H)

    def padded_slice(arr, start, rows):
        lead = max(0, -start)
        s0 = max(0, start)
        e0 = min(n, start + rows)
        return jnp.pad(arr[s0:e0], ((lead, rows - (e0 - s0) - lead), (0, 0)))

    max_d = int(math.floor(math.log2(n)))
    top_d = 9
    Bp = 512
    B2 = 2 * Bp

    s_l, sp_l, L_l, C_l, G_l = [], [], [], [], []
    prev_Lp = 0
    for d in range(max_d, top_d, -1):
        s = 2 ** d - 1
        e = min(2 ** (d + 1) - 1, n)
        L = e - s
        Lp = (L + 1) // 2
        s_l.append(s)
        sp_l.append(2 ** (d - 1) - 1)
        L_l.append(L)
        C_l.append(prev_Lp)
        G_l.append(-(-Lp // Bp))
        prev_Lp = Lp
    nlv = len(s_l)
    start = [0]
    for g in G_l:
        start.append(start[-1] + g)
    n_steps = start[-1]
    start = start[:-1]
    cap8 = [0] + [max(0, G_l[k - 1] * Bp - B2) // 8 for k in range(1, nlv)]
    buf_rows = max(g * Bp for g in G_l)
    tbl = {
        "start": start,
        "s8": [(s - _SH) // 8 for s in s_l],
        "sp8": [(sp - _SH) // 8 for sp in sp_l],
        "L": L_l,
        "C": C_l,
        "cap8": cap8,
    }
    tail_step = G_l[0] - 1
    tb = s_l[0] - _SH + tail_step * B2
    x_t = padded_slice(embed, tb, B2 + 8)
    ss_t = padded_slice(structure_sum, tb, B2 + 8)
    sc_t = padded_slice(structure_c, tb, B2 + 8)
    xp_t = padded_slice(embed, sp_l[0] - _SH + tail_step * Bp, Bp + 8)

    R_pair = jnp.equal(jnp.arange(B2)[None, :] // 2,
                       jnp.arange(Bp)[:, None]).astype(f32)

    hbm = pl.BlockSpec(memory_space=pltpu.MemorySpace.HBM)

    def full(shape):
        return pl.BlockSpec(shape, lambda i: (0,) * len(shape))

    dh, dc, hacc = pl.pallas_call(
        functools.partial(_big_body, H, in_dim, Bp, tbl, tail_step, n_steps),
        grid=(n_steps,),
        in_specs=[hbm, hbm, hbm, hbm, hbm, hbm, hbm,
                  full((Bp, B2)),
                  full((in_dim, 3 * H)), full((H, 3 * H)), full((1, 3 * H)),
                  full((in_dim, H)), full((H, H)), full((1, H))],
        out_specs=[full((Bp, H)), full((Bp, H)), full((1, H))],
        out_shape=[jax.ShapeDtypeStruct((Bp, H), f32),
                   jax.ShapeDtypeStruct((Bp, H), f32),
                   jax.ShapeDtypeStruct((1, H), f32)],
        scratch_shapes=[pltpu.VMEM((3, B2 + 8, in_dim), f32),
                        pltpu.VMEM((3, B2 + 8, H), f32),
                        pltpu.VMEM((3, B2 + 8, H), f32),
                        pltpu.VMEM((3, Bp + 8, in_dim), f32),
                        pltpu.VMEM((2, buf_rows, H), f32),
                        pltpu.VMEM((2, buf_rows, H), f32),
                        pltpu.SemaphoreType.DMA((3, 4))],
        compiler_params=pltpu.CompilerParams(
            dimension_semantics=("arbitrary",)),
    )(embed, structure_sum, structure_c, x_t, ss_t, sc_t, xp_t,
      R_pair, W_iou, U_iou, b_iou2, W_f, U_f, b_f2)

    n_top = 2 ** (top_d + 1) - 1
    x_a = _pad_rows(embed[0:n_top], n_top + 1)
    ss_a = _pad_rows(structure_sum[0:n_top], n_top + 1)
    sc_a = _pad_rows(structure_c[0:n_top], n_top + 1)
    mu, lv_ = pl.pallas_call(
        functools.partial(_top_body, H, top_d),
        out_shape=[jax.ShapeDtypeStruct((1, H // 2), f32),
                   jax.ShapeDtypeStruct((1, H // 2), f32)],
    )(x_a, ss_a, sc_a, dh, dc, W_iou, U_iou, b_iou2, W_f, U_f, b_f2, hacc)
    return (mu, lv_)


# speedup vs baseline: 1.0218x; 1.0218x over previous
import functools
import math

import jax
import jax.numpy as jnp
from jax.experimental import pallas as pl
from jax.experimental.pallas import tpu as pltpu

_SH = 7


def _sel(table, idx):
    v = jnp.int32(table[0])
    for k in range(1, len(table)):
        v = jnp.where(idx >= k, jnp.int32(table[k]), v)
    return v


def _big_body(H, in_dim, Bp, tbl, tail_step, n_steps,
              embed, ss_h, sc_h, xt, sst, sct, xpt,
              R_pair, W_iou, U_iou, b_iou, W_f, U_f, b_f,
              dh_fin, dc_fin, hacc_out,
              x_buf, ss_buf, sc_buf, xp_buf, dh_buf, dc_buf, sems):
    B2 = 2 * Bp

    def params(step):
        lv = jnp.int32(0)
        for st in tbl["start"][1:]:
            lv = lv + jnp.where(step >= st, 1, 0).astype(jnp.int32)
        j = step - _sel(tbl["start"], lv)
        return lv, j

    def copy_in(step, slot):
        lv, j = params(step)
        cb8 = _sel(tbl["s8"], lv) + j * (B2 // 8)
        pb8 = _sel(tbl["sp8"], lv) + j * (Bp // 8)
        cb = pl.multiple_of(cb8 * 8, 8)
        pb = pl.multiple_of(pb8 * 8, 8)

        @pl.when(step != tail_step)
        def _():
            pltpu.make_async_copy(embed.at[pl.ds(cb, B2 + 8), :],
                                  x_buf.at[slot], sems.at[slot, 0]).start()
            pltpu.make_async_copy(ss_h.at[pl.ds(cb, B2 + 8), :],
                                  ss_buf.at[slot], sems.at[slot, 1]).start()
            pltpu.make_async_copy(sc_h.at[pl.ds(cb, B2 + 8), :],
                                  sc_buf.at[slot], sems.at[slot, 2]).start()
            pltpu.make_async_copy(embed.at[pl.ds(pb, Bp + 8), :],
                                  xp_buf.at[slot], sems.at[slot, 3]).start()

        @pl.when(step == tail_step)
        def _():
            pltpu.make_async_copy(xt.at[:, :], x_buf.at[slot],
                                  sems.at[slot, 0]).start()
            pltpu.make_async_copy(sst.at[:, :], ss_buf.at[slot],
                                  sems.at[slot, 1]).start()
            pltpu.make_async_copy(sct.at[:, :], sc_buf.at[slot],
                                  sems.at[slot, 2]).start()
            pltpu.make_async_copy(xpt.at[:, :], xp_buf.at[slot],
                                  sems.at[slot, 3]).start()

    def wait_one(slot, k, buf, ref, rows_):
        pltpu.make_async_copy(ref.at[pl.ds(0, rows_), :],
                              buf.at[slot], sems.at[slot, k]).wait()

    i = pl.program_id(0)
    slot = jax.lax.rem(i, 2)

    @pl.when(i == 0)
    def _():
        copy_in(0, 0)

    @pl.when(i + 1 < n_steps)
    def _():
        copy_in(i + 1, jax.lax.rem(i + 1, 2))

    lv, j = params(i)
    p = jax.lax.rem(lv, 2)
    L_rem = _sel(tbl["L"], lv) - j * B2
    C_rem = _sel(tbl["C"], lv) - j * B2
    rd8 = jnp.minimum(j * (B2 // 8), _sel(tbl["cap8"], lv))
    rd = pl.multiple_of(rd8 * 8, 8)
    wr = pl.multiple_of(j * Bp, Bp)

    rows = jax.lax.broadcasted_iota(jnp.int32, (B2, 1), 0)
    m = rows < L_rem
    m_c = rows < C_rem

    wait_one(slot, 0, x_buf, embed, B2 + 8)
    x = x_buf[slot][_SH:_SH + B2]
    xW = jnp.dot(x, W_iou[:, :], preferred_element_type=jnp.float32)

    wait_one(slot, 1, ss_buf, ss_h, B2 + 8)
    wait_one(slot, 2, sc_buf, sc_h, B2 + 8)
    hsum = ss_buf[slot][_SH:_SH + B2]
    cin = sc_buf[slot][_SH:_SH + B2]
    hsum = hsum + jnp.where(m_c, dh_buf[1 - p, pl.ds(rd, B2), :], 0.0)
    cin = cin + jnp.where(m_c, dc_buf[1 - p, pl.ds(rd, B2), :], 0.0)
    iou = (xW + b_iou[:, :]
           + jnp.dot(hsum, U_iou[:, :], preferred_element_type=jnp.float32))
    c = jax.nn.sigmoid(iou[:, :H]) * jnp.tanh(iou[:, 2 * H:]) + cin
    h = jax.nn.sigmoid(iou[:, H:2 * H]) * jnp.tanh(c)
    h = jnp.where(m, h, 0.0)

    wait_one(slot, 3, xp_buf, embed, Bp + 8)
    xpW = jnp.dot(xp_buf[slot][_SH:_SH + Bp], W_f[:, :],
                  preferred_element_type=jnp.float32) + b_f[:, :]
    xpW2 = jnp.broadcast_to(xpW[:, None, :], (Bp, 2, H)).reshape(B2, H)
    f = jax.nn.sigmoid(xpW2 + jnp.dot(h, U_f[:, :],
                                      preferred_element_type=jnp.float32))
    fc = jnp.where(m, f * c, 0.0)
    dh_t = jnp.dot(R_pair[:, :], h, preferred_element_type=jnp.float32)
    dc_t = jnp.dot(R_pair[:, :], fc, preferred_element_type=jnp.float32)
    dh_buf[p, pl.ds(wr, Bp), :] = dh_t
    dc_buf[p, pl.ds(wr, Bp), :] = dc_t
    part = jnp.sum(h, axis=0, keepdims=True)

    @pl.when(i == 0)
    def _():
        hacc_out[:, :] = part

    @pl.when(i > 0)
    def _():
        hacc_out[:, :] = hacc_out[:, :] + part

    @pl.when(i == n_steps - 1)
    def _():
        dh_fin[:, :] = dh_t
        dc_fin[:, :] = dc_t


def _top_body(H, top_d, x_all, ss_all, sc_all, dh_in, dc_in,
              W_iou, U_iou, b_iou, W_f, U_f, b_f, hacc_in, mu_out, lv_out):
    Wi = W_iou[:, :]
    Ui = U_iou[:, :]
    bi = b_iou[:, :]
    Wf = W_f[:, :]
    Uf = U_f[:, :]
    bf = b_f[:, :]
    xv = x_all[:, :]
    ssv = ss_all[:, :]
    scv = sc_all[:, :]
    hacc = hacc_in[:, :]
    dh = dh_in[:, :]
    dc = dc_in[:, :]

    for d in range(top_d, -1, -1):
        s = 2 ** d - 1
        L = 2 ** d
        x_l = xv[s:s + L]
        hs = ssv[s:s + L] + dh[:L]
        cn = scv[s:s + L] + dc[:L]
        iou = (jnp.dot(x_l, Wi, preferred_element_type=jnp.float32) + bi
               + jnp.dot(hs, Ui, preferred_element_type=jnp.float32))
        c = jax.nn.sigmoid(iou[:, :H]) * jnp.tanh(iou[:, 2 * H:]) + cn
        h = jax.nn.sigmoid(iou[:, H:2 * H]) * jnp.tanh(c)
        hacc = hacc + jnp.sum(h, axis=0, keepdims=True)
        if d == 0:
            break
        Lp = L // 2
        sp = 2 ** (d - 1) - 1
        xpW = jnp.dot(xv[sp:sp + Lp], Wf,
                      preferred_element_type=jnp.float32) + bf
        xpW2 = jnp.broadcast_to(xpW[:, None, :], (Lp, 2, H)).reshape(L, H)
        f = jax.nn.sigmoid(xpW2 + jnp.dot(h, Uf,
                                          preferred_element_type=jnp.float32))
        dh = h.reshape(Lp, 2, H).sum(axis=1)
        dc = (f * c).reshape(Lp, 2, H).sum(axis=1)

    mu_out[:, :] = hacc[:, :H // 2]
    lv_out[:, :] = jnp.tanh(hacc[:, H // 2:])


def _pad_rows(a, rows):
    if a.shape[0] == rows:
        return a
    return jnp.pad(a, ((0, rows - a.shape[0]), (0, 0)))


def kernel(embed, edge_index, structure_sum, structure_c,
           W_iou, U_iou, b_iou, W_f, U_f, b_f):
    del edge_index
    n = embed.shape[0]
    in_dim = embed.shape[1]
    H = U_f.shape[0]
    f32 = jnp.float32

    b_iou2 = b_iou.reshape(1, 3 * H)
    b_f2 = b_f.reshape(1, H)

    def padded_slice(arr, start, rows):
        lead = max(0, -start)
        s0 = max(0, start)
        e0 = min(n, start + rows)
        return jnp.pad(arr[s0:e0], ((lead, rows - (e0 - s0) - lead), (0, 0)))

    max_d = int(math.floor(math.log2(n)))
    top_d = 9
    Bp = 512
    B2 = 2 * Bp

    s_l, sp_l, L_l, C_l, G_l = [], [], [], [], []
    prev_Lp = 0
    for d in range(max_d, top_d, -1):
        s = 2 ** d - 1
        e = min(2 ** (d + 1) - 1, n)
        L = e - s
        Lp = (L + 1) // 2
        s_l.append(s)
        sp_l.append(2 ** (d - 1) - 1)
        L_l.append(L)
        C_l.append(prev_Lp)
        G_l.append(-(-Lp // Bp))
        prev_Lp = Lp
    nlv = len(s_l)
    start = [0]
    for g in G_l:
        start.append(start[-1] + g)
    n_steps = start[-1]
    start = start[:-1]
    cap8 = [0] + [max(0, G_l[k - 1] * Bp - B2) // 8 for k in range(1, nlv)]
    buf_rows = max(g * Bp for g in G_l)
    tbl = {
        "start": start,
        "s8": [(s - _SH) // 8 for s in s_l],
        "sp8": [(sp - _SH) // 8 for sp in sp_l],
        "L": L_l,
        "C": C_l,
        "cap8": cap8,
    }
    tail_step = G_l[0] - 1
    tb = s_l[0] - _SH + tail_step * B2
    x_t = padded_slice(embed, tb, B2 + 8)
    ss_t = padded_slice(structure_sum, tb, B2 + 8)
    sc_t = padded_slice(structure_c, tb, B2 + 8)
    xp_t = padded_slice(embed, sp_l[0] - _SH + tail_step * Bp, Bp + 8)

    R_pair = jnp.equal(jnp.arange(B2)[None, :] // 2,
                       jnp.arange(Bp)[:, None]).astype(f32)

    hbm = pl.BlockSpec(memory_space=pltpu.MemorySpace.HBM)

    def full(shape):
        return pl.BlockSpec(shape, lambda i: (0,) * len(shape))

    dh, dc, hacc = pl.pallas_call(
        functools.partial(_big_body, H, in_dim, Bp, tbl, tail_step, n_steps),
        grid=(n_steps,),
        in_specs=[hbm, hbm, hbm, hbm, hbm, hbm, hbm,
                  full((Bp, B2)),
                  full((in_dim, 3 * H)), full((H, 3 * H)), full((1, 3 * H)),
                  full((in_dim, H)), full((H, H)), full((1, H))],
        out_specs=[full((Bp, H)), full((Bp, H)), full((1, H))],
        out_shape=[jax.ShapeDtypeStruct((Bp, H), f32),
                   jax.ShapeDtypeStruct((Bp, H), f32),
                   jax.ShapeDtypeStruct((1, H), f32)],
        scratch_shapes=[pltpu.VMEM((2, B2 + 8, in_dim), f32),
                        pltpu.VMEM((2, B2 + 8, H), f32),
                        pltpu.VMEM((2, B2 + 8, H), f32),
                        pltpu.VMEM((2, Bp + 8, in_dim), f32),
                        pltpu.VMEM((2, buf_rows, H), f32),
                        pltpu.VMEM((2, buf_rows, H), f32),
                        pltpu.SemaphoreType.DMA((2, 4))],
        compiler_params=pltpu.CompilerParams(
            dimension_semantics=("arbitrary",)),
    )(embed, structure_sum, structure_c, x_t, ss_t, sc_t, xp_t,
      R_pair, W_iou, U_iou, b_iou2, W_f, U_f, b_f2)

    n_top = 2 ** (top_d + 1) - 1
    x_a = _pad_rows(embed[0:n_top], n_top + 1)
    ss_a = _pad_rows(structure_sum[0:n_top], n_top + 1)
    sc_a = _pad_rows(structure_c[0:n_top], n_top + 1)
    mu, lv_ = pl.pallas_call(
        functools.partial(_top_body, H, top_d),
        out_shape=[jax.ShapeDtypeStruct((1, H // 2), f32),
                   jax.ShapeDtypeStruct((1, H // 2), f32)],
    )(x_a, ss_a, sc_a, dh, dc, W_iou, U_iou, b_iou2, W_f, U_f, b_f2, hacc)
    return (mu, lv_)


# speedup vs baseline: 1.0869x; 1.0638x over previous
import functools
import math

import jax
import jax.numpy as jnp
from jax.experimental import pallas as pl
from jax.experimental.pallas import tpu as pltpu

_SH = 7


def _sel(table, idx):
    v = jnp.int32(table[0])
    for k in range(1, len(table)):
        v = jnp.where(idx >= k, jnp.int32(table[k]), v)
    return v


def _big_body(H, in_dim, Bp, tbl, tail_step, n_steps,
              embed, ss_h, sc_h, xt, sst, sct, xpt,
              R_pair, W_iou, U_iou, b_iou, W_f, U_f, b_f,
              dh_fin, dc_fin, hacc_out,
              x_buf, ss_buf, sc_buf, xp_buf, dh_buf, dc_buf, sems):
    B2 = 2 * Bp

    def params(step):
        lv = jnp.int32(0)
        for st in tbl["start"][1:]:
            lv = lv + jnp.where(step >= st, 1, 0).astype(jnp.int32)
        j = step - _sel(tbl["start"], lv)
        return lv, j

    def copy_in(step, slot):
        lv, j = params(step)
        cb8 = _sel(tbl["s8"], lv) + j * (B2 // 8)
        pb8 = _sel(tbl["sp8"], lv) + j * (Bp // 8)
        cb = pl.multiple_of(cb8 * 8, 8)
        pb = pl.multiple_of(pb8 * 8, 8)

        @pl.when(step != tail_step)
        def _():
            pltpu.make_async_copy(embed.at[pl.ds(cb, B2 + 8), :],
                                  x_buf.at[slot], sems.at[slot, 0]).start()
            pltpu.make_async_copy(ss_h.at[pl.ds(cb, B2 + 8), :],
                                  ss_buf.at[slot], sems.at[slot, 1]).start()
            pltpu.make_async_copy(sc_h.at[pl.ds(cb, B2 + 8), :],
                                  sc_buf.at[slot], sems.at[slot, 2]).start()
            pltpu.make_async_copy(embed.at[pl.ds(pb, Bp + 8), :],
                                  xp_buf.at[slot], sems.at[slot, 3]).start()

        @pl.when(step == tail_step)
        def _():
            pltpu.make_async_copy(xt.at[:, :], x_buf.at[slot],
                                  sems.at[slot, 0]).start()
            pltpu.make_async_copy(sst.at[:, :], ss_buf.at[slot],
                                  sems.at[slot, 1]).start()
            pltpu.make_async_copy(sct.at[:, :], sc_buf.at[slot],
                                  sems.at[slot, 2]).start()
            pltpu.make_async_copy(xpt.at[:, :], xp_buf.at[slot],
                                  sems.at[slot, 3]).start()

    def wait_one(slot, k, buf, ref, rows_):
        pltpu.make_async_copy(ref.at[pl.ds(0, rows_), :],
                              buf.at[slot], sems.at[slot, k]).wait()

    i = pl.program_id(0)
    slot = jax.lax.rem(i, 2)

    @pl.when(i == 0)
    def _():
        copy_in(0, 0)

    @pl.when(i + 1 < n_steps)
    def _():
        copy_in(i + 1, jax.lax.rem(i + 1, 2))

    lv, j = params(i)
    p = jax.lax.rem(lv, 2)
    L_rem = _sel(tbl["L"], lv) - j * B2
    C_rem = _sel(tbl["C"], lv) - j * B2
    rd8 = jnp.minimum(j * (B2 // 8), _sel(tbl["cap8"], lv))
    rd = pl.multiple_of(rd8 * 8, 8)
    wr = pl.multiple_of(j * Bp, Bp)

    rows = jax.lax.broadcasted_iota(jnp.int32, (B2, 1), 0)
    m = rows < L_rem
    m_c = rows < C_rem

    wait_one(slot, 0, x_buf, embed, B2 + 8)
    wait_one(slot, 1, ss_buf, ss_h, B2 + 8)
    wait_one(slot, 2, sc_buf, sc_h, B2 + 8)
    wait_one(slot, 3, xp_buf, embed, Bp + 8)

    hsum = ss_buf[slot][_SH:_SH + B2]
    cin = sc_buf[slot][_SH:_SH + B2]
    hsum = hsum + jnp.where(m_c, dh_buf[1 - p, pl.ds(rd, B2), :], 0.0)
    cin = cin + jnp.where(m_c, dc_buf[1 - p, pl.ds(rd, B2), :], 0.0)
    x = x_buf[slot][_SH:_SH + B2]
    iou = (jnp.dot(x, W_iou[:, :], preferred_element_type=jnp.float32)
           + b_iou[:, :]
           + jnp.dot(hsum, U_iou[:, :], preferred_element_type=jnp.float32))
    c = jax.nn.sigmoid(iou[:, :H]) * jnp.tanh(iou[:, 2 * H:]) + cin
    h = jax.nn.sigmoid(iou[:, H:2 * H]) * jnp.tanh(c)
    h = jnp.where(m, h, 0.0)

    xpW = jnp.dot(xp_buf[slot][_SH:_SH + Bp], W_f[:, :],
                  preferred_element_type=jnp.float32) + b_f[:, :]
    xpW2 = jnp.broadcast_to(xpW[:, None, :], (Bp, 2, H)).reshape(B2, H)
    f = jax.nn.sigmoid(xpW2 + jnp.dot(h, U_f[:, :],
                                      preferred_element_type=jnp.float32))
    fc = jnp.where(m, f * c, 0.0)
    dh_t = jnp.dot(R_pair[:, :], h, preferred_element_type=jnp.float32)
    dc_t = jnp.dot(R_pair[:, :], fc, preferred_element_type=jnp.float32)
    dh_buf[p, pl.ds(wr, Bp), :] = dh_t
    dc_buf[p, pl.ds(wr, Bp), :] = dc_t
    part = jnp.sum(h, axis=0, keepdims=True)

    @pl.when(i == 0)
    def _():
        hacc_out[:, :] = part

    @pl.when(i > 0)
    def _():
        hacc_out[:, :] = hacc_out[:, :] + part

    @pl.when(i == n_steps - 1)
    def _():
        dh_fin[:, :] = dh_t
        dc_fin[:, :] = dc_t


def _top_body(H, top_d, x_all, ss_all, sc_all, dh_in, dc_in,
              W_iou, U_iou, b_iou, W_f, U_f, b_f, hacc_in, mu_out, lv_out):
    Wi = W_iou[:, :]
    Ui = U_iou[:, :]
    bi = b_iou[:, :]
    Wf = W_f[:, :]
    Uf = U_f[:, :]
    bf = b_f[:, :]
    xv = x_all[:, :]
    ssv = ss_all[:, :]
    scv = sc_all[:, :]
    hacc = hacc_in[:, :]
    dh = dh_in[:, :]
    dc = dc_in[:, :]

    for d in range(top_d, -1, -1):
        s = 2 ** d - 1
        L = 2 ** d
        x_l = xv[s:s + L]
        hs = ssv[s:s + L] + dh[:L]
        cn = scv[s:s + L] + dc[:L]
        iou = (jnp.dot(x_l, Wi, preferred_element_type=jnp.float32) + bi
               + jnp.dot(hs, Ui, preferred_element_type=jnp.float32))
        c = jax.nn.sigmoid(iou[:, :H]) * jnp.tanh(iou[:, 2 * H:]) + cn
        h = jax.nn.sigmoid(iou[:, H:2 * H]) * jnp.tanh(c)
        hacc = hacc + jnp.sum(h, axis=0, keepdims=True)
        if d == 0:
            break
        Lp = L // 2
        sp = 2 ** (d - 1) - 1
        xpW = jnp.dot(xv[sp:sp + Lp], Wf,
                      preferred_element_type=jnp.float32) + bf
        xpW2 = jnp.broadcast_to(xpW[:, None, :], (Lp, 2, H)).reshape(L, H)
        f = jax.nn.sigmoid(xpW2 + jnp.dot(h, Uf,
                                          preferred_element_type=jnp.float32))
        dh = h.reshape(Lp, 2, H).sum(axis=1)
        dc = (f * c).reshape(Lp, 2, H).sum(axis=1)

    mu_out[:, :] = hacc[:, :H // 2]
    lv_out[:, :] = jnp.tanh(hacc[:, H // 2:])


def _pad_rows(a, rows):
    if a.shape[0] == rows:
        return a
    return jnp.pad(a, ((0, rows - a.shape[0]), (0, 0)))


def kernel(embed, edge_index, structure_sum, structure_c,
           W_iou, U_iou, b_iou, W_f, U_f, b_f):
    del edge_index
    n = embed.shape[0]
    in_dim = embed.shape[1]
    H = U_f.shape[0]
    f32 = jnp.float32

    b_iou2 = b_iou.reshape(1, 3 * H)
    b_f2 = b_f.reshape(1, H)

    def padded_slice(arr, start, rows):
        lead = max(0, -start)
        s0 = max(0, start)
        e0 = min(n, start + rows)
        return jnp.pad(arr[s0:e0], ((lead, rows - (e0 - s0) - lead), (0, 0)))

    max_d = int(math.floor(math.log2(n)))
    top_d = 9
    Bp = 512
    B2 = 2 * Bp

    s_l, sp_l, L_l, C_l, G_l = [], [], [], [], []
    prev_Lp = 0
    for d in range(max_d, top_d, -1):
        s = 2 ** d - 1
        e = min(2 ** (d + 1) - 1, n)
        L = e - s
        Lp = (L + 1) // 2
        s_l.append(s)
        sp_l.append(2 ** (d - 1) - 1)
        L_l.append(L)
        C_l.append(prev_Lp)
        G_l.append(-(-Lp // Bp))
        prev_Lp = Lp
    nlv = len(s_l)
    start = [0]
    for g in G_l:
        start.append(start[-1] + g)
    n_steps = start[-1]
    start = start[:-1]
    cap8 = [0] + [max(0, G_l[k - 1] * Bp - B2) // 8 for k in range(1, nlv)]
    buf_rows = max(g * Bp for g in G_l)
    tbl = {
        "start": start,
        "s8": [(s - _SH) // 8 for s in s_l],
        "sp8": [(sp - _SH) // 8 for sp in sp_l],
        "L": L_l,
        "C": C_l,
        "cap8": cap8,
    }
    tail_step = G_l[0] - 1
    tb = s_l[0] - _SH + tail_step * B2
    x_t = padded_slice(embed, tb, B2 + 8)
    ss_t = padded_slice(structure_sum, tb, B2 + 8)
    sc_t = padded_slice(structure_c, tb, B2 + 8)
    xp_t = padded_slice(embed, sp_l[0] - _SH + tail_step * Bp, Bp + 8)

    R_pair = jnp.equal(jnp.arange(B2)[None, :] // 2,
                       jnp.arange(Bp)[:, None]).astype(f32)

    hbm = pl.BlockSpec(memory_space=pltpu.MemorySpace.HBM)

    def full(shape):
        return pl.BlockSpec(shape, lambda i: (0,) * len(shape))

    dh, dc, hacc = pl.pallas_call(
        functools.partial(_big_body, H, in_dim, Bp, tbl, tail_step, n_steps),
        grid=(n_steps,),
        in_specs=[hbm, hbm, hbm, hbm, hbm, hbm, hbm,
                  full((Bp, B2)),
                  full((in_dim, 3 * H)), full((H, 3 * H)), full((1, 3 * H)),
                  full((in_dim, H)), full((H, H)), full((1, H))],
        out_specs=[full((Bp, H)), full((Bp, H)), full((1, H))],
        out_shape=[jax.ShapeDtypeStruct((Bp, H), f32),
                   jax.ShapeDtypeStruct((Bp, H), f32),
                   jax.ShapeDtypeStruct((1, H), f32)],
        scratch_shapes=[pltpu.VMEM((2, B2 + 8, in_dim), f32),
                        pltpu.VMEM((2, B2 + 8, H), f32),
                        pltpu.VMEM((2, B2 + 8, H), f32),
                        pltpu.VMEM((2, Bp + 8, in_dim), f32),
                        pltpu.VMEM((2, buf_rows, H), f32),
                        pltpu.VMEM((2, buf_rows, H), f32),
                        pltpu.SemaphoreType.DMA((2, 4))],
        compiler_params=pltpu.CompilerParams(
            dimension_semantics=("arbitrary",)),
    )(embed, structure_sum, structure_c, x_t, ss_t, sc_t, xp_t,
      R_pair, W_iou, U_iou, b_iou2, W_f, U_f, b_f2)

    n_top = 2 ** (top_d + 1) - 1
    x_a = _pad_rows(embed[0:n_top], n_top + 1)
    ss_a = _pad_rows(structure_sum[0:n_top], n_top + 1)
    sc_a = _pad_rows(structure_c[0:n_top], n_top + 1)
    mu, lv_ = pl.pallas_call(
        functools.partial(_top_body, H, top_d),
        out_shape=[jax.ShapeDtypeStruct((1, H // 2), f32),
                   jax.ShapeDtypeStruct((1, H // 2), f32)],
    )(x_a, ss_a, sc_a, dh, dc, W_iou, U_iou, b_iou2, W_f, U_f, b_f2, hacc)
    return (mu, lv_)
